# baseline (speedup 1.0000x reference)
import functools

import jax
import jax.numpy as jnp
from jax import lax
from jax.experimental import pallas as pl
from jax.experimental.pallas import tpu as pltpu
from jax.experimental.pallas import tpu_sc as plsc

_N = 43008
_H = 64
_W = 2 * _H
_E = 688128
_NSC = 2
_NT = 16
_NW = _NSC * _NT
_CH = 128
_K = _E // (_NT * _CH)
_KD = _E // (_NW * _CH)
_QTR = _N // 4
_TRASH = _CH
_ACC_ROWS = _QTR + _TRASH
_DUMP_PT = _QTR // _NT
_ZERO_PT = _ACC_ROWS // _NT
_ZCH = 85
_RB = 1024
_G = _N // _RB
_B = 128


def _deg_body(edge_hbm, deg_hbm, colstage, hist):
  c = lax.axis_index("c")
  s = lax.axis_index("s")
  wid = s * _NSC + c
  pltpu.sync_copy(edge_hbm.at[1, pl.ds(wid * _KD * _CH, _KD * _CH)], colstage)

  zero16 = jnp.zeros((16,), jnp.float32)

  @pl.loop(0, _N // 16)
  def _(i):
    hist[pl.ds(i * 16, 16)] = zero16

  @pl.loop(0, _KD)
  def _(k):
    for j in range(8):
      idx = colstage[pl.ds(k * _CH + j * 16, 16)]
      cnt, last = plsc.scan_count(idx)
      plsc.addupdate_scatter(hist, [idx], cnt.astype(jnp.float32), mask=last)

  pltpu.sync_copy(hist, deg_hbm.at[wid])


_STRIP = 16
_NSTRIP = _K // _STRIP
_SEL = _STRIP * _CH + _CH


_SCH = _STRIP * _CH


def _sweep_body(edge_hbm, y_hbm, zeros_hbm, acc_hbm,
                rbuf, cbuf, rsel, dsel, didx, gbuf, acc, sem, sem_i,
                sem_s0, sem_s1):
  c = lax.axis_index("c")
  t = lax.axis_index("s")
  iota = lax.iota(jnp.int32, 16)
  ebase = t * _K * _CH

  for q in range(2):
    lo = (c * 2 + q) * _QTR

    for z in range(_ZERO_PT // _ZCH):
      pltpu.sync_copy(zeros_hbm, acc.at[pl.ds(t * _ZERO_PT + z * _ZCH, _ZCH)])
    pltpu.async_copy(edge_hbm.at[0, pl.ds(ebase, _SCH)], rbuf, sem_i)
    pltpu.async_copy(edge_hbm.at[1, pl.ds(ebase, _SCH)], cbuf, sem_i)
    plsc.subcore_barrier()

    @pl.loop(0, _NSTRIP)
    def _(s):
      pltpu.make_async_copy(edge_hbm.at[0, pl.ds(0, _SCH)], rbuf,
                            sem_i).wait()
      pltpu.make_async_copy(edge_hbm.at[1, pl.ds(0, _SCH)], cbuf,
                            sem_i).wait()

      def _filter(k, cur):
        for j in range(8):
          d = cbuf[pl.ds(k * _CH + j * 16, 16)]
          r = rbuf[pl.ds(k * _CH + j * 16, 16)]
          m = (d >= lo) & (d < lo + _QTR)
          incl = plsc.cumsum(m.astype(jnp.int32))
          pos = cur + incl - 1
          plsc.store_scatter(dsel, [pos], d - lo, mask=m)
          plsc.store_scatter(rsel, [pos], r, mask=m)
          cur = cur + incl[15]
        return cur

      cur = pl.loop(0, _STRIP, init_carry=0)(_filter)

      @pl.when(s < _NSTRIP - 1)
      def _():
        pltpu.async_copy(
            edge_hbm.at[0, pl.ds(ebase + (s + 1) * _SCH, _SCH)], rbuf, sem_i)
        pltpu.async_copy(
            edge_hbm.at[1, pl.ds(ebase + (s + 1) * _SCH, _SCH)], cbuf, sem_i)

      for j in range(8):
        rows = j * 16 + iota
        plsc.store_scatter(rsel, [cur + rows], rows)
        plsc.store_scatter(dsel, [cur + rows], _QTR + rows)

      @pl.when(cur > 0)
      def _():
        pltpu.async_copy(y_hbm.at[rsel.at[pl.ds(0, _CH)]], gbuf.at[0], sem)

      nr = (cur + _CH - 1) // _CH

      @pl.loop(0, _SEL // _CH)
      def _(r):
        @pl.when(r * _CH < cur)
        def _():
          p = lax.rem(r, 2)

          @pl.when((r >= 2) & (p == 0))
          def _():
            pltpu.make_async_copy(gbuf.at[1], acc.at[didx.at[1]],
                                  sem_s1).wait()

          @pl.when(p == 1)
          def _():
            pltpu.make_async_copy(gbuf.at[0], acc.at[didx.at[0]],
                                  sem_s0).wait()

          pltpu.make_async_copy(y_hbm.at[pl.ds(0, _CH)], gbuf.at[0],
                                sem).wait()

          @pl.when((r + 1) * _CH < cur)
          def _():
            pltpu.async_copy(y_hbm.at[rsel.at[pl.ds((r + 1) * _CH, _CH)]],
                             gbuf.at[1 - p], sem)

          @pl.when(p == 0)
          def _():
            for j in range(8):
              didx[0, pl.ds(j * 16, 16)] = dsel[pl.ds(r * _CH + j * 16, 16)]
            pltpu.async_copy(gbuf.at[0], acc.at[didx.at[0]], sem_s0, add=True)

          @pl.when(p == 1)
          def _():
            for j in range(8):
              didx[1, pl.ds(j * 16, 16)] = dsel[pl.ds(r * _CH + j * 16, 16)]
            pltpu.async_copy(gbuf.at[1], acc.at[didx.at[1]], sem_s1, add=True)

      @pl.when(lax.rem(nr, 2) == 1)
      def _():
        pltpu.make_async_copy(gbuf.at[0], acc.at[didx.at[0]], sem_s0).wait()

      @pl.when((nr > 0) & (lax.rem(nr, 2) == 0))
      def _():
        pltpu.make_async_copy(gbuf.at[1], acc.at[didx.at[1]], sem_s1).wait()

    plsc.subcore_barrier()
    pltpu.sync_copy(acc.at[pl.ds(t * _DUMP_PT, _DUMP_PT)],
                    acc_hbm.at[pl.ds(lo + t * _DUMP_PT, _DUMP_PT)])
    plsc.subcore_barrier()


def _deg_col(deg_ref):
  return lax.dot_general(deg_ref[...], jnp.ones((_NW, 1), jnp.float32),
                         (((0,), (0,)), ((), ())),
                         preferred_element_type=jnp.float32)


def _pre_body(amp_ref, ph_ref, pos_ref, deg_ref, wa_ref, wp_ref, y_ref):
  a = amp_ref[...] + pos_ref[...]
  p = ph_ref[...] + pos_ref[...]
  dis = lax.rsqrt(_deg_col(deg_ref) + 1.0)
  y_ref[:, 0:_H] = dis * jnp.dot(a, wa_ref[...],
                                 preferred_element_type=jnp.float32)
  y_ref[:, _H:_W] = dis * jnp.dot(p, wp_ref[...],
                                  preferred_element_type=jnp.float32)


def _post_body(acc_ref, y_ref, deg_ref, batch_ref, b1a_ref, b1p_ref,
               cw_ref, cb_ref, lwt_ref, lb_ref, s_ref,
               out_ref, pool_a, pool_p, cnt):
  i = pl.program_id(0)
  dis = lax.rsqrt(_deg_col(deg_ref) + 1.0)
  u = acc_ref[...] + y_ref[...]
  ha = jnp.maximum(dis * u[:, 0:_H] + b1a_ref[...], 0.0)
  hp = jnp.maximum(dis * u[:, _H:_W] + b1p_ref[...], 0.0)
  onehot = (batch_ref[0] == lax.broadcasted_iota(jnp.int32, (_RB, _B), 1)
            ).astype(jnp.float32)
  dims = (((0,), (0,)), ((), ()))
  pa = lax.dot_general(onehot, ha, dims, preferred_element_type=jnp.float32)
  pp = lax.dot_general(onehot, hp, dims, preferred_element_type=jnp.float32)
  c = lax.dot_general(onehot, jnp.ones((_RB, _B), jnp.float32), dims,
                      preferred_element_type=jnp.float32)

  @pl.when(i == 0)
  def _():
    pool_a[...] = pa
    pool_p[...] = pp
    cnt[...] = c

  @pl.when(i > 0)
  def _():
    pool_a[...] += pa
    pool_p[...] += pp
    cnt[...] += c

  @pl.when(i == _G - 1)
  def _():
    cm = jnp.maximum(cnt[...][:, 0:_H], 1.0)
    x = (pool_a[...] / cm) * cw_ref[:, 0:1] + (pool_p[...] / cm) * cw_ref[:, 1:2]
    x = x + cb_ref[...]
    logits = jnp.dot(x, lwt_ref[...], preferred_element_type=jnp.float32)
    logits = (logits + lb_ref[...]) * s_ref[...]
    mx = jnp.max(logits, axis=1, keepdims=True)
    e = jnp.exp(logits - mx)
    out_ref[...] = e / jnp.sum(e, axis=1, keepdims=True)


@functools.cache
def _sc_kernels():
  mesh = plsc.VectorSubcoreMesh(core_axis_name="c", subcore_axis_name="s",
                                num_cores=_NSC, num_subcores=_NT)
  deg_kernel = pl.kernel(
      _deg_body,
      out_type=jax.ShapeDtypeStruct((_NW, _N), jnp.float32),
      mesh=mesh,
      compiler_params=pltpu.CompilerParams(needs_layout_passes=False),
      scratch_types=[
          pltpu.VMEM((_KD * _CH,), jnp.int32),
          pltpu.VMEM((_N,), jnp.float32),
      ],
  )
  sweep_kernel = pl.kernel(
      _sweep_body,
      out_type=jax.ShapeDtypeStruct((_N, _W), jnp.float32),
      mesh=mesh,
      compiler_params=pltpu.CompilerParams(needs_layout_passes=False),
      scratch_types=[
          pltpu.VMEM((_SCH,), jnp.int32),
          pltpu.VMEM((_SCH,), jnp.int32),
          pltpu.VMEM((_SEL,), jnp.int32),
          pltpu.VMEM((_SEL,), jnp.int32),
          pltpu.VMEM((2, _CH), jnp.int32),
          pltpu.VMEM((2, _CH, _W), jnp.float32),
          pltpu.VMEM_SHARED((_ACC_ROWS, _W), jnp.float32),
          pltpu.SemaphoreType.DMA,
          pltpu.SemaphoreType.DMA,
          pltpu.SemaphoreType.DMA,
          pltpu.SemaphoreType.DMA,
      ],
  )
  return deg_kernel, sweep_kernel


_pre_kernel = pl.pallas_call(
    _pre_body,
    grid=(_G,),
    in_specs=[
        pl.BlockSpec((_RB, _H), lambda i: (i, 0)),
        pl.BlockSpec((_RB, _H), lambda i: (i, 0)),
        pl.BlockSpec((_RB, _H), lambda i: (i, 0)),
        pl.BlockSpec((_NW, _RB), lambda i: (0, i)),
        pl.BlockSpec((_H, _H), lambda i: (0, 0)),
        pl.BlockSpec((_H, _H), lambda i: (0, 0)),
    ],
    out_specs=pl.BlockSpec((_RB, _W), lambda i: (i, 0)),
    out_shape=jax.ShapeDtypeStruct((_N, _W), jnp.float32),
)

_post_kernel = pl.pallas_call(
    _post_body,
    grid=(_G,),
    in_specs=[
        pl.BlockSpec((_RB, _W), lambda i: (i, 0)),
        pl.BlockSpec((_RB, _W), lambda i: (i, 0)),
        pl.BlockSpec((_NW, _RB), lambda i: (0, i)),
        pl.BlockSpec((1, _RB, 1), lambda i: (i, 0, 0)),
        pl.BlockSpec((1, _H), lambda i: (0, 0)),
        pl.BlockSpec((1, _H), lambda i: (0, 0)),
        pl.BlockSpec((1, 2), lambda i: (0, 0)),
        pl.BlockSpec((1, 1), lambda i: (0, 0)),
        pl.BlockSpec((_H, 10), lambda i: (0, 0)),
        pl.BlockSpec((1, 10), lambda i: (0, 0)),
        pl.BlockSpec((1, 1), lambda i: (0, 0)),
    ],
    out_specs=pl.BlockSpec((_B, 10), lambda i: (0, 0)),
    out_shape=jax.ShapeDtypeStruct((_B, 10), jnp.float32),
    scratch_shapes=[
        pltpu.VMEM((_B, _H), jnp.float32),
        pltpu.VMEM((_B, _H), jnp.float32),
        pltpu.VMEM((_B, _B), jnp.float32),
    ],
)


def kernel(amp, ph, pos_embed, W1_amp, b1_amp, W1_ph, b1_ph, conv1d_w,
           conv1d_b, lin_W, lin_b, edge_index, batch, batch_size):
  bs = amp.shape[0]
  amp_r = amp.reshape(_N, _H)
  ph_r = ph.reshape(_N, _H)
  pos_r = pos_embed.reshape(_N, _H)
  zeros_fill = jnp.zeros((_ZCH, _W), jnp.float32)

  deg_kernel, sweep_kernel = _sc_kernels()
  deg32 = deg_kernel(edge_index)
  y = _pre_kernel(amp_r, ph_r, pos_r, deg32, W1_amp.T, W1_ph.T)
  acc = sweep_kernel(edge_index, y, zeros_fill)

  batch3 = batch.reshape(_G, _RB, 1)
  scale = (jnp.float32(batch_size) / bs).reshape(1, 1)
  return _post_kernel(
      acc, y, deg32, batch3,
      b1_amp.reshape(1, _H), b1_ph.reshape(1, _H),
      conv1d_w.reshape(1, 2), conv1d_b.reshape(1, 1),
      lin_W.T, lin_b.reshape(1, 10), scale)


# speedup vs baseline: 1.0726x; 1.0726x over previous
import functools

import jax
import jax.numpy as jnp
from jax import lax
from jax.experimental import pallas as pl
from jax.experimental.pallas import tpu as pltpu
from jax.experimental.pallas import tpu_sc as plsc

_N = 43008
_H = 64
_W = 2 * _H
_E = 688128
_NSC = 2
_NT = 16
_NW = _NSC * _NT
_CH = 128
_K = _E // (_NT * _CH)
_KD = _E // (_NW * _CH)
_QTR = _N // 4
_TRASH = _CH
_ACC_ROWS = _QTR + _TRASH
_DUMP_PT = _QTR // _NT
_ZERO_PT = _ACC_ROWS // _NT
_ZCH = 85
_RB = 1024
_G = _N // _RB
_B = 128


def _deg_body(edge_hbm, deg_hbm, colstage, hist):
  c = lax.axis_index("c")
  s = lax.axis_index("s")
  wid = s * _NSC + c
  pltpu.sync_copy(edge_hbm.at[1, pl.ds(wid * _KD * _CH, _KD * _CH)], colstage)

  zero16 = jnp.zeros((16,), jnp.float32)

  @pl.loop(0, _N // 16)
  def _(i):
    hist[pl.ds(i * 16, 16)] = zero16

  @pl.loop(0, _KD)
  def _(k):
    for j in range(8):
      idx = colstage[pl.ds(k * _CH + j * 16, 16)]
      cnt, last = plsc.scan_count(idx)
      plsc.addupdate_scatter(hist, [idx], cnt.astype(jnp.float32), mask=last)

  pltpu.sync_copy(hist, deg_hbm.at[wid])


_STRIP = 16
_NSTRIP = _K // _STRIP
_SEL = _STRIP * _CH + _CH


_SCH = _STRIP * _CH


def _sweep_body(edge_hbm, y_hbm, zeros_hbm, acc_hbm,
                rbuf, cbuf, rsel, dsel, didx, gbuf, acc, sem, sem_i,
                sem_s0, sem_s1):
  c = lax.axis_index("c")
  t = lax.axis_index("s")
  iota = lax.iota(jnp.int32, 16)
  ebase = t * _K * _CH

  for q in range(2):
    lo = (c * 2 + q) * _QTR

    for z in range(_ZERO_PT // _ZCH):
      pltpu.sync_copy(zeros_hbm, acc.at[pl.ds(t * _ZERO_PT + z * _ZCH, _ZCH)])
    pltpu.async_copy(edge_hbm.at[0, pl.ds(ebase, _SCH)], rbuf, sem_i)
    pltpu.async_copy(edge_hbm.at[1, pl.ds(ebase, _SCH)], cbuf, sem_i)
    plsc.subcore_barrier()

    @pl.loop(0, _NSTRIP)
    def _(s):
      pltpu.make_async_copy(edge_hbm.at[0, pl.ds(0, _SCH)], rbuf,
                            sem_i).wait()
      pltpu.make_async_copy(edge_hbm.at[1, pl.ds(0, _SCH)], cbuf,
                            sem_i).wait()

      def _filter(k, cur):
        ds_, rs_, ms_, incls = [], [], [], []
        for j in range(8):
          d = cbuf[pl.ds(k * _CH + j * 16, 16)]
          r = rbuf[pl.ds(k * _CH + j * 16, 16)]
          m = (d >= lo) & (d < lo + _QTR)
          ds_.append(d)
          rs_.append(r)
          ms_.append(m)
          incls.append(plsc.cumsum(m.astype(jnp.int32)))
        for j in range(8):
          pos = cur + incls[j] - 1
          plsc.store_scatter(dsel, [pos], ds_[j] - lo, mask=ms_[j])
          plsc.store_scatter(rsel, [pos], rs_[j], mask=ms_[j])
          cur = cur + incls[j][15]
        return cur

      cur = pl.loop(0, _STRIP, init_carry=0)(_filter)

      @pl.when(s < _NSTRIP - 1)
      def _():
        pltpu.async_copy(
            edge_hbm.at[0, pl.ds(ebase + (s + 1) * _SCH, _SCH)], rbuf, sem_i)
        pltpu.async_copy(
            edge_hbm.at[1, pl.ds(ebase + (s + 1) * _SCH, _SCH)], cbuf, sem_i)

      for j in range(8):
        rows = j * 16 + iota
        plsc.store_scatter(rsel, [cur + rows], rows)
        plsc.store_scatter(dsel, [cur + rows], _QTR + rows)

      @pl.when(cur > 0)
      def _():
        pltpu.async_copy(y_hbm.at[rsel.at[pl.ds(0, _CH)]], gbuf.at[0], sem)

      nr = (cur + _CH - 1) // _CH

      @pl.loop(0, _SEL // _CH)
      def _(r):
        @pl.when(r * _CH < cur)
        def _():
          p = lax.rem(r, 2)

          @pl.when((r >= 2) & (p == 0))
          def _():
            pltpu.make_async_copy(gbuf.at[1], acc.at[didx.at[1]],
                                  sem_s1).wait()

          @pl.when(p == 1)
          def _():
            pltpu.make_async_copy(gbuf.at[0], acc.at[didx.at[0]],
                                  sem_s0).wait()

          pltpu.make_async_copy(y_hbm.at[pl.ds(0, _CH)], gbuf.at[0],
                                sem).wait()

          @pl.when((r + 1) * _CH < cur)
          def _():
            pltpu.async_copy(y_hbm.at[rsel.at[pl.ds((r + 1) * _CH, _CH)]],
                             gbuf.at[1 - p], sem)

          @pl.when(p == 0)
          def _():
            for j in range(8):
              didx[0, pl.ds(j * 16, 16)] = dsel[pl.ds(r * _CH + j * 16, 16)]
            pltpu.async_copy(gbuf.at[0], acc.at[didx.at[0]], sem_s0, add=True)

          @pl.when(p == 1)
          def _():
            for j in range(8):
              didx[1, pl.ds(j * 16, 16)] = dsel[pl.ds(r * _CH + j * 16, 16)]
            pltpu.async_copy(gbuf.at[1], acc.at[didx.at[1]], sem_s1, add=True)

      @pl.when(lax.rem(nr, 2) == 1)
      def _():
        pltpu.make_async_copy(gbuf.at[0], acc.at[didx.at[0]], sem_s0).wait()

      @pl.when((nr > 0) & (lax.rem(nr, 2) == 0))
      def _():
        pltpu.make_async_copy(gbuf.at[1], acc.at[didx.at[1]], sem_s1).wait()

    plsc.subcore_barrier()
    pltpu.sync_copy(acc.at[pl.ds(t * _DUMP_PT, _DUMP_PT)],
                    acc_hbm.at[pl.ds(lo + t * _DUMP_PT, _DUMP_PT)])
    plsc.subcore_barrier()


def _deg_col(deg_ref):
  return lax.dot_general(deg_ref[...], jnp.ones((_NW, 1), jnp.float32),
                         (((0,), (0,)), ((), ())),
                         preferred_element_type=jnp.float32)


def _pre_body(amp_ref, ph_ref, pos_ref, deg_ref, wa_ref, wp_ref, y_ref):
  a = amp_ref[...] + pos_ref[...]
  p = ph_ref[...] + pos_ref[...]
  dis = lax.rsqrt(_deg_col(deg_ref) + 1.0)
  y_ref[:, 0:_H] = dis * jnp.dot(a, wa_ref[...],
                                 preferred_element_type=jnp.float32)
  y_ref[:, _H:_W] = dis * jnp.dot(p, wp_ref[...],
                                  preferred_element_type=jnp.float32)


def _post_body(acc_ref, y_ref, deg_ref, batch_ref, b1a_ref, b1p_ref,
               cw_ref, cb_ref, lwt_ref, lb_ref, s_ref,
               out_ref, pool_a, pool_p, cnt):
  i = pl.program_id(0)
  dis = lax.rsqrt(_deg_col(deg_ref) + 1.0)
  u = acc_ref[...] + y_ref[...]
  ha = jnp.maximum(dis * u[:, 0:_H] + b1a_ref[...], 0.0)
  hp = jnp.maximum(dis * u[:, _H:_W] + b1p_ref[...], 0.0)
  onehot = (batch_ref[0] == lax.broadcasted_iota(jnp.int32, (_RB, _B), 1)
            ).astype(jnp.float32)
  dims = (((0,), (0,)), ((), ()))
  pa = lax.dot_general(onehot, ha, dims, preferred_element_type=jnp.float32)
  pp = lax.dot_general(onehot, hp, dims, preferred_element_type=jnp.float32)
  c = lax.dot_general(onehot, jnp.ones((_RB, _B), jnp.float32), dims,
                      preferred_element_type=jnp.float32)

  @pl.when(i == 0)
  def _():
    pool_a[...] = pa
    pool_p[...] = pp
    cnt[...] = c

  @pl.when(i > 0)
  def _():
    pool_a[...] += pa
    pool_p[...] += pp
    cnt[...] += c

  @pl.when(i == _G - 1)
  def _():
    cm = jnp.maximum(cnt[...][:, 0:_H], 1.0)
    x = (pool_a[...] / cm) * cw_ref[:, 0:1] + (pool_p[...] / cm) * cw_ref[:, 1:2]
    x = x + cb_ref[...]
    logits = jnp.dot(x, lwt_ref[...], preferred_element_type=jnp.float32)
    logits = (logits + lb_ref[...]) * s_ref[...]
    mx = jnp.max(logits, axis=1, keepdims=True)
    e = jnp.exp(logits - mx)
    out_ref[...] = e / jnp.sum(e, axis=1, keepdims=True)


@functools.cache
def _sc_kernels():
  mesh = plsc.VectorSubcoreMesh(core_axis_name="c", subcore_axis_name="s",
                                num_cores=_NSC, num_subcores=_NT)
  deg_kernel = pl.kernel(
      _deg_body,
      out_type=jax.ShapeDtypeStruct((_NW, _N), jnp.float32),
      mesh=mesh,
      compiler_params=pltpu.CompilerParams(needs_layout_passes=False),
      scratch_types=[
          pltpu.VMEM((_KD * _CH,), jnp.int32),
          pltpu.VMEM((_N,), jnp.float32),
      ],
  )
  sweep_kernel = pl.kernel(
      _sweep_body,
      out_type=jax.ShapeDtypeStruct((_N, _W), jnp.float32),
      mesh=mesh,
      compiler_params=pltpu.CompilerParams(needs_layout_passes=False),
      scratch_types=[
          pltpu.VMEM((_SCH,), jnp.int32),
          pltpu.VMEM((_SCH,), jnp.int32),
          pltpu.VMEM((_SEL,), jnp.int32),
          pltpu.VMEM((_SEL,), jnp.int32),
          pltpu.VMEM((2, _CH), jnp.int32),
          pltpu.VMEM((2, _CH, _W), jnp.float32),
          pltpu.VMEM_SHARED((_ACC_ROWS, _W), jnp.float32),
          pltpu.SemaphoreType.DMA,
          pltpu.SemaphoreType.DMA,
          pltpu.SemaphoreType.DMA,
          pltpu.SemaphoreType.DMA,
      ],
  )
  return deg_kernel, sweep_kernel


_pre_kernel = pl.pallas_call(
    _pre_body,
    grid=(_G,),
    in_specs=[
        pl.BlockSpec((_RB, _H), lambda i: (i, 0)),
        pl.BlockSpec((_RB, _H), lambda i: (i, 0)),
        pl.BlockSpec((_RB, _H), lambda i: (i, 0)),
        pl.BlockSpec((_NW, _RB), lambda i: (0, i)),
        pl.BlockSpec((_H, _H), lambda i: (0, 0)),
        pl.BlockSpec((_H, _H), lambda i: (0, 0)),
    ],
    out_specs=pl.BlockSpec((_RB, _W), lambda i: (i, 0)),
    out_shape=jax.ShapeDtypeStruct((_N, _W), jnp.float32),
)

_post_kernel = pl.pallas_call(
    _post_body,
    grid=(_G,),
    in_specs=[
        pl.BlockSpec((_RB, _W), lambda i: (i, 0)),
        pl.BlockSpec((_RB, _W), lambda i: (i, 0)),
        pl.BlockSpec((_NW, _RB), lambda i: (0, i)),
        pl.BlockSpec((1, _RB, 1), lambda i: (i, 0, 0)),
        pl.BlockSpec((1, _H), lambda i: (0, 0)),
        pl.BlockSpec((1, _H), lambda i: (0, 0)),
        pl.BlockSpec((1, 2), lambda i: (0, 0)),
        pl.BlockSpec((1, 1), lambda i: (0, 0)),
        pl.BlockSpec((_H, 10), lambda i: (0, 0)),
        pl.BlockSpec((1, 10), lambda i: (0, 0)),
        pl.BlockSpec((1, 1), lambda i: (0, 0)),
    ],
    out_specs=pl.BlockSpec((_B, 10), lambda i: (0, 0)),
    out_shape=jax.ShapeDtypeStruct((_B, 10), jnp.float32),
    scratch_shapes=[
        pltpu.VMEM((_B, _H), jnp.float32),
        pltpu.VMEM((_B, _H), jnp.float32),
        pltpu.VMEM((_B, _B), jnp.float32),
    ],
)


def kernel(amp, ph, pos_embed, W1_amp, b1_amp, W1_ph, b1_ph, conv1d_w,
           conv1d_b, lin_W, lin_b, edge_index, batch, batch_size):
  bs = amp.shape[0]
  amp_r = amp.reshape(_N, _H)
  ph_r = ph.reshape(_N, _H)
  pos_r = pos_embed.reshape(_N, _H)
  zeros_fill = jnp.zeros((_ZCH, _W), jnp.float32)

  deg_kernel, sweep_kernel = _sc_kernels()
  deg32 = deg_kernel(edge_index)
  y = _pre_kernel(amp_r, ph_r, pos_r, deg32, W1_amp.T, W1_ph.T)
  acc = sweep_kernel(edge_index, y, zeros_fill)

  batch3 = batch.reshape(_G, _RB, 1)
  scale = (jnp.float32(batch_size) / bs).reshape(1, 1)
  return _post_kernel(
      acc, y, deg32, batch3,
      b1_amp.reshape(1, _H), b1_ph.reshape(1, _H),
      conv1d_w.reshape(1, 2), conv1d_b.reshape(1, 1),
      lin_W.T, lin_b.reshape(1, 10), scale)


# speedup vs baseline: 1.1085x; 1.0334x over previous
import functools

import jax
import jax.numpy as jnp
from jax import lax
from jax.experimental import pallas as pl
from jax.experimental.pallas import tpu as pltpu
from jax.experimental.pallas import tpu_sc as plsc

_N = 43008
_H = 64
_W = 2 * _H
_E = 688128
_NSC = 2
_NT = 16
_NW = _NSC * _NT
_CH = 128
_K = _E // (_NT * _CH)
_KD = _E // (_NW * _CH)
_QTR = _N // 4
_TRASH = 96
_ACC_ROWS = _QTR + _TRASH
_DUMP_PT = _QTR // _NT
_ZERO_PT = _ACC_ROWS // _NT
_ZCH = 113
_RB = 1024
_G = _N // _RB
_B = 128


def _deg_body(edge_hbm, deg_hbm, colstage, hist):
  c = lax.axis_index("c")
  s = lax.axis_index("s")
  wid = s * _NSC + c
  pltpu.sync_copy(edge_hbm.at[1, pl.ds(wid * _KD * _CH, _KD * _CH)], colstage)

  zero16 = jnp.zeros((16,), jnp.float32)

  @pl.loop(0, _N // 16)
  def _(i):
    hist[pl.ds(i * 16, 16)] = zero16

  @pl.loop(0, _KD)
  def _(k):
    for j in range(8):
      idx = colstage[pl.ds(k * _CH + j * 16, 16)]
      cnt, last = plsc.scan_count(idx)
      plsc.addupdate_scatter(hist, [idx], cnt.astype(jnp.float32), mask=last)

  pltpu.sync_copy(hist, deg_hbm.at[wid])


_STRIP = 21
_NSTRIP = _K // _STRIP
_SEL = _STRIP * _CH + _CH


_SCH = _STRIP * _CH


def _sweep_body(edge_hbm, y_hbm, zeros_hbm, acc_hbm,
                rbuf, cbuf, rsel, dsel, didx, gbuf, acc, sem, sem_i,
                sem_s0, sem_s1):
  c = lax.axis_index("c")
  t = lax.axis_index("s")
  iota = lax.iota(jnp.int32, 16)
  ebase = t * _K * _CH

  for q in range(2):
    lo = (c * 2 + q) * _QTR

    for z in range(_ZERO_PT // _ZCH):
      pltpu.sync_copy(zeros_hbm, acc.at[pl.ds(t * _ZERO_PT + z * _ZCH, _ZCH)])
    pltpu.async_copy(edge_hbm.at[0, pl.ds(ebase, _SCH)], rbuf, sem_i)
    pltpu.async_copy(edge_hbm.at[1, pl.ds(ebase, _SCH)], cbuf, sem_i)
    plsc.subcore_barrier()

    @pl.loop(0, _NSTRIP)
    def _(s):
      pltpu.make_async_copy(edge_hbm.at[0, pl.ds(0, _SCH)], rbuf,
                            sem_i).wait()
      pltpu.make_async_copy(edge_hbm.at[1, pl.ds(0, _SCH)], cbuf,
                            sem_i).wait()

      def _filter(k, cur):
        ds_, rs_, ms_, incls = [], [], [], []
        for j in range(8):
          d = cbuf[pl.ds(k * _CH + j * 16, 16)]
          r = rbuf[pl.ds(k * _CH + j * 16, 16)]
          m = (d >= lo) & (d < lo + _QTR)
          ds_.append(d)
          rs_.append(r)
          ms_.append(m)
          incls.append(plsc.cumsum(m.astype(jnp.int32)))
        for j in range(8):
          pos = cur + incls[j] - 1
          plsc.store_scatter(dsel, [pos], ds_[j] - lo, mask=ms_[j])
          plsc.store_scatter(rsel, [pos], rs_[j], mask=ms_[j])
          cur = cur + incls[j][15]
        return cur

      cur = pl.loop(0, _STRIP, init_carry=0)(_filter)

      @pl.when(s < _NSTRIP - 1)
      def _():
        pltpu.async_copy(
            edge_hbm.at[0, pl.ds(ebase + (s + 1) * _SCH, _SCH)], rbuf, sem_i)
        pltpu.async_copy(
            edge_hbm.at[1, pl.ds(ebase + (s + 1) * _SCH, _SCH)], cbuf, sem_i)

      for j in range(8):
        rows = j * 16 + iota
        plsc.store_scatter(rsel, [cur + rows], rows)
        plsc.store_scatter(dsel, [cur + rows], _QTR + lax.rem(rows, _TRASH))

      @pl.when(cur > 0)
      def _():
        pltpu.async_copy(y_hbm.at[rsel.at[pl.ds(0, _CH)]], gbuf.at[0], sem)

      nr = (cur + _CH - 1) // _CH

      @pl.loop(0, _SEL // _CH)
      def _(r):
        @pl.when(r * _CH < cur)
        def _():
          p = lax.rem(r, 2)

          @pl.when((r >= 2) & (p == 0))
          def _():
            pltpu.make_async_copy(gbuf.at[1], acc.at[didx.at[1]],
                                  sem_s1).wait()

          @pl.when(p == 1)
          def _():
            pltpu.make_async_copy(gbuf.at[0], acc.at[didx.at[0]],
                                  sem_s0).wait()

          pltpu.make_async_copy(y_hbm.at[pl.ds(0, _CH)], gbuf.at[0],
                                sem).wait()

          @pl.when((r + 1) * _CH < cur)
          def _():
            pltpu.async_copy(y_hbm.at[rsel.at[pl.ds((r + 1) * _CH, _CH)]],
                             gbuf.at[1 - p], sem)

          @pl.when(p == 0)
          def _():
            for j in range(8):
              didx[0, pl.ds(j * 16, 16)] = dsel[pl.ds(r * _CH + j * 16, 16)]
            pltpu.async_copy(gbuf.at[0], acc.at[didx.at[0]], sem_s0, add=True)

          @pl.when(p == 1)
          def _():
            for j in range(8):
              didx[1, pl.ds(j * 16, 16)] = dsel[pl.ds(r * _CH + j * 16, 16)]
            pltpu.async_copy(gbuf.at[1], acc.at[didx.at[1]], sem_s1, add=True)

      @pl.when(lax.rem(nr, 2) == 1)
      def _():
        pltpu.make_async_copy(gbuf.at[0], acc.at[didx.at[0]], sem_s0).wait()

      @pl.when((nr > 0) & (lax.rem(nr, 2) == 0))
      def _():
        pltpu.make_async_copy(gbuf.at[1], acc.at[didx.at[1]], sem_s1).wait()

    plsc.subcore_barrier()
    pltpu.sync_copy(acc.at[pl.ds(t * _DUMP_PT, _DUMP_PT)],
                    acc_hbm.at[pl.ds(lo + t * _DUMP_PT, _DUMP_PT)])
    plsc.subcore_barrier()


def _deg_col(deg_ref):
  return lax.dot_general(deg_ref[...], jnp.ones((_NW, 1), jnp.float32),
                         (((0,), (0,)), ((), ())),
                         preferred_element_type=jnp.float32)


def _pre_body(amp_ref, ph_ref, pos_ref, deg_ref, wa_ref, wp_ref, y_ref):
  a = amp_ref[...] + pos_ref[...]
  p = ph_ref[...] + pos_ref[...]
  dis = lax.rsqrt(_deg_col(deg_ref) + 1.0)
  y_ref[:, 0:_H] = dis * jnp.dot(a, wa_ref[...],
                                 preferred_element_type=jnp.float32)
  y_ref[:, _H:_W] = dis * jnp.dot(p, wp_ref[...],
                                  preferred_element_type=jnp.float32)


def _post_body(acc_ref, y_ref, deg_ref, batch_ref, b1a_ref, b1p_ref,
               cw_ref, cb_ref, lwt_ref, lb_ref, s_ref,
               out_ref, pool_a, pool_p, cnt):
  i = pl.program_id(0)
  dis = lax.rsqrt(_deg_col(deg_ref) + 1.0)
  u = acc_ref[...] + y_ref[...]
  ha = jnp.maximum(dis * u[:, 0:_H] + b1a_ref[...], 0.0)
  hp = jnp.maximum(dis * u[:, _H:_W] + b1p_ref[...], 0.0)
  onehot = (batch_ref[0] == lax.broadcasted_iota(jnp.int32, (_RB, _B), 1)
            ).astype(jnp.float32)
  dims = (((0,), (0,)), ((), ()))
  pa = lax.dot_general(onehot, ha, dims, preferred_element_type=jnp.float32)
  pp = lax.dot_general(onehot, hp, dims, preferred_element_type=jnp.float32)
  c = lax.dot_general(onehot, jnp.ones((_RB, _B), jnp.float32), dims,
                      preferred_element_type=jnp.float32)

  @pl.when(i == 0)
  def _():
    pool_a[...] = pa
    pool_p[...] = pp
    cnt[...] = c

  @pl.when(i > 0)
  def _():
    pool_a[...] += pa
    pool_p[...] += pp
    cnt[...] += c

  @pl.when(i == _G - 1)
  def _():
    cm = jnp.maximum(cnt[...][:, 0:_H], 1.0)
    x = (pool_a[...] / cm) * cw_ref[:, 0:1] + (pool_p[...] / cm) * cw_ref[:, 1:2]
    x = x + cb_ref[...]
    logits = jnp.dot(x, lwt_ref[...], preferred_element_type=jnp.float32)
    logits = (logits + lb_ref[...]) * s_ref[...]
    mx = jnp.max(logits, axis=1, keepdims=True)
    e = jnp.exp(logits - mx)
    out_ref[...] = e / jnp.sum(e, axis=1, keepdims=True)


@functools.cache
def _sc_kernels():
  mesh = plsc.VectorSubcoreMesh(core_axis_name="c", subcore_axis_name="s",
                                num_cores=_NSC, num_subcores=_NT)
  deg_kernel = pl.kernel(
      _deg_body,
      out_type=jax.ShapeDtypeStruct((_NW, _N), jnp.float32),
      mesh=mesh,
      compiler_params=pltpu.CompilerParams(needs_layout_passes=False),
      scratch_types=[
          pltpu.VMEM((_KD * _CH,), jnp.int32),
          pltpu.VMEM((_N,), jnp.float32),
      ],
  )
  sweep_kernel = pl.kernel(
      _sweep_body,
      out_type=jax.ShapeDtypeStruct((_N, _W), jnp.float32),
      mesh=mesh,
      compiler_params=pltpu.CompilerParams(needs_layout_passes=False),
      scratch_types=[
          pltpu.VMEM((_SCH,), jnp.int32),
          pltpu.VMEM((_SCH,), jnp.int32),
          pltpu.VMEM((_SEL,), jnp.int32),
          pltpu.VMEM((_SEL,), jnp.int32),
          pltpu.VMEM((2, _CH), jnp.int32),
          pltpu.VMEM((2, _CH, _W), jnp.float32),
          pltpu.VMEM_SHARED((_ACC_ROWS, _W), jnp.float32),
          pltpu.SemaphoreType.DMA,
          pltpu.SemaphoreType.DMA,
          pltpu.SemaphoreType.DMA,
          pltpu.SemaphoreType.DMA,
      ],
  )
  return deg_kernel, sweep_kernel


_pre_kernel = pl.pallas_call(
    _pre_body,
    grid=(_G,),
    in_specs=[
        pl.BlockSpec((_RB, _H), lambda i: (i, 0)),
        pl.BlockSpec((_RB, _H), lambda i: (i, 0)),
        pl.BlockSpec((_RB, _H), lambda i: (i, 0)),
        pl.BlockSpec((_NW, _RB), lambda i: (0, i)),
        pl.BlockSpec((_H, _H), lambda i: (0, 0)),
        pl.BlockSpec((_H, _H), lambda i: (0, 0)),
    ],
    out_specs=pl.BlockSpec((_RB, _W), lambda i: (i, 0)),
    out_shape=jax.ShapeDtypeStruct((_N, _W), jnp.float32),
)

_post_kernel = pl.pallas_call(
    _post_body,
    grid=(_G,),
    in_specs=[
        pl.BlockSpec((_RB, _W), lambda i: (i, 0)),
        pl.BlockSpec((_RB, _W), lambda i: (i, 0)),
        pl.BlockSpec((_NW, _RB), lambda i: (0, i)),
        pl.BlockSpec((1, _RB, 1), lambda i: (i, 0, 0)),
        pl.BlockSpec((1, _H), lambda i: (0, 0)),
        pl.BlockSpec((1, _H), lambda i: (0, 0)),
        pl.BlockSpec((1, 2), lambda i: (0, 0)),
        pl.BlockSpec((1, 1), lambda i: (0, 0)),
        pl.BlockSpec((_H, 10), lambda i: (0, 0)),
        pl.BlockSpec((1, 10), lambda i: (0, 0)),
        pl.BlockSpec((1, 1), lambda i: (0, 0)),
    ],
    out_specs=pl.BlockSpec((_B, 10), lambda i: (0, 0)),
    out_shape=jax.ShapeDtypeStruct((_B, 10), jnp.float32),
    scratch_shapes=[
        pltpu.VMEM((_B, _H), jnp.float32),
        pltpu.VMEM((_B, _H), jnp.float32),
        pltpu.VMEM((_B, _B), jnp.float32),
    ],
)


def kernel(amp, ph, pos_embed, W1_amp, b1_amp, W1_ph, b1_ph, conv1d_w,
           conv1d_b, lin_W, lin_b, edge_index, batch, batch_size):
  bs = amp.shape[0]
  amp_r = amp.reshape(_N, _H)
  ph_r = ph.reshape(_N, _H)
  pos_r = pos_embed.reshape(_N, _H)
  zeros_fill = jnp.zeros((_ZCH, _W), jnp.float32)

  deg_kernel, sweep_kernel = _sc_kernels()
  deg32 = deg_kernel(edge_index)
  y = _pre_kernel(amp_r, ph_r, pos_r, deg32, W1_amp.T, W1_ph.T)
  acc = sweep_kernel(edge_index, y, zeros_fill)

  batch3 = batch.reshape(_G, _RB, 1)
  scale = (jnp.float32(batch_size) / bs).reshape(1, 1)
  return _post_kernel(
      acc, y, deg32, batch3,
      b1_amp.reshape(1, _H), b1_ph.reshape(1, _H),
      conv1d_w.reshape(1, 2), conv1d_b.reshape(1, 1),
      lin_W.T, lin_b.reshape(1, 10), scale)
